# baseline (speedup 1.0000x reference)
import jax
import jax.numpy as jnp
from jax import lax
from jax.experimental import pallas as pl
from jax.experimental.pallas import tpu as pltpu

B = 64
C = 768
D = 24 * 24

WAVE = 16
NWAVE = C // WAVE


def _tc_shuffle(idx_smem, x_hbm, out_hbm, sem0, sem1):
    sems = (sem0, sem1)

    def issue(w, s):
        for k in range(WAVE):
            j = w * WAVE + k
            src = idx_smem[j]
            pltpu.make_async_copy(
                x_hbm.at[:, pl.ds(src, 1), :],
                out_hbm.at[:, pl.ds(j, 1), :],
                sems[s],
            ).start()

    def drain(w, s):
        for k in range(WAVE):
            j = w * WAVE + k
            pltpu.make_async_copy(
                x_hbm.at[:, pl.ds(0, 1), :],
                out_hbm.at[:, pl.ds(j, 1), :],
                sems[s],
            ).wait()

    issue(0, 0)

    def body(p, carry):
        w = p * 2
        issue(w + 1, 1)
        drain(w, 0)

        @pl.when(w + 2 < NWAVE)
        def _():
            issue(w + 2, 0)

        drain(w + 1, 1)
        return carry

    lax.fori_loop(0, NWAVE // 2, body, 0)


@jax.jit
def _shuffle(x, forward_shuffle_idx):
    xr = x.reshape(B, C, D)
    out = pl.pallas_call(
        _tc_shuffle,
        grid=(),
        in_specs=[
            pl.BlockSpec(memory_space=pltpu.SMEM),
            pl.BlockSpec(memory_space=pl.ANY),
        ],
        out_specs=pl.BlockSpec(memory_space=pl.ANY),
        out_shape=jax.ShapeDtypeStruct((B, C, D), jnp.float32),
        scratch_shapes=[pltpu.SemaphoreType.DMA, pltpu.SemaphoreType.DMA],
    )(forward_shuffle_idx, xr)
    return out.reshape(B, C, 24, 24)


def kernel(x, forward_shuffle_idx):
    return (_shuffle(x, forward_shuffle_idx), 0)


# speedup vs baseline: 3.0144x; 3.0144x over previous
import jax
import jax.numpy as jnp
from jax.experimental import pallas as pl
from jax.experimental.pallas import tpu as pltpu

B = 64
C = 768
D = 24 * 24


def _copy_body(idx_ref, in_ref, out_ref):
    out_ref[...] = in_ref[...]


@jax.jit
def _shuffle(x, forward_shuffle_idx):
    grid_spec = pltpu.PrefetchScalarGridSpec(
        num_scalar_prefetch=1,
        grid=(C,),
        in_specs=[
            pl.BlockSpec((B, 1, 24, 24),
                         lambda j, idx_ref: (0, idx_ref[j], 0, 0)),
        ],
        out_specs=pl.BlockSpec((B, 1, 24, 24),
                               lambda j, idx_ref: (0, j, 0, 0)),
    )
    out = pl.pallas_call(
        _copy_body,
        grid_spec=grid_spec,
        out_shape=jax.ShapeDtypeStruct((B, C, 24, 24), jnp.float32),
    )(forward_shuffle_idx, x)
    return out


def kernel(x, forward_shuffle_idx):
    return (_shuffle(x, forward_shuffle_idx), 0)


# speedup vs baseline: 12.8139x; 4.2509x over previous
import jax
import jax.numpy as jnp
from jax import lax
from jax.experimental import pallas as pl
from jax.experimental.pallas import tpu as pltpu

B = 64
C = 768
D = 24 * 24
BB = 4


def _permute_body(idx_ref, in_ref, out_ref):
    def body(j, carry):
        src = idx_ref[j]
        out_ref[:, pl.ds(j, 1), :] = in_ref[:, pl.ds(src, 1), :]
        return carry

    lax.fori_loop(0, C, body, 0, unroll=8)


@jax.jit
def _shuffle(x, forward_shuffle_idx):
    xr = x.reshape(B, C, D)
    grid_spec = pltpu.PrefetchScalarGridSpec(
        num_scalar_prefetch=1,
        grid=(B // BB,),
        in_specs=[
            pl.BlockSpec((BB, C, D), lambda i, idx_ref: (i, 0, 0)),
        ],
        out_specs=pl.BlockSpec((BB, C, D), lambda i, idx_ref: (i, 0, 0)),
    )
    out = pl.pallas_call(
        _permute_body,
        grid_spec=grid_spec,
        out_shape=jax.ShapeDtypeStruct((B, C, D), jnp.float32),
    )(forward_shuffle_idx, xr)
    return out.reshape(B, C, 24, 24)


def kernel(x, forward_shuffle_idx):
    return (_shuffle(x, forward_shuffle_idx), 0)
